# baseline (speedup 1.0000x reference)
import functools

import jax
import jax.numpy as jnp
from jax import lax
from jax.experimental import pallas as pl
from jax.experimental.pallas import tpu as pltpu
from jax.experimental.pallas import tpu_sc as plsc

M, B, D = 100000, 16384, 64
LANES = 16
NC, NS = 2, 16
NW = NC * NS
CHUNK = 3136
M_PAD = NW * CHUNK
GROUPS = B // LANES
N_PER_W = B // NW
ICH = 128
NCH = N_PER_W // ICH


def _mesh():
    return plsc.VectorSubcoreMesh(core_axis_name="c", subcore_axis_name="s")


_SC_PARAMS = pltpu.CompilerParams(
    needs_layout_passes=False, use_tc_tiling_on_sc=False)


@functools.partial(
    pl.kernel,
    out_type=jax.ShapeDtypeStruct((M_PAD,), jnp.int32),
    mesh=_mesh(),
    compiler_params=_SC_PARAMS,
    scratch_types=[
        pltpu.VMEM((B,), jnp.int32),
        pltpu.VMEM((CHUNK,), jnp.int32),
    ],
)
def _build_table(idx_hbm, t_hbm, idx_v, tbl_v):
    wid = lax.axis_index("s") * NC + lax.axis_index("c")
    base = wid * CHUNK

    def fill(i, carry):
        tbl_v[pl.ds(i * LANES, LANES)] = jnp.full((LANES,), -1, jnp.int32)
        return carry

    lax.fori_loop(0, CHUNK // LANES, fill, 0)

    pltpu.sync_copy(idx_hbm, idx_v)
    lane = lax.iota(jnp.int32, LANES)

    def body(g, carry):
        iv = idx_v[pl.ds(g * LANES, LANES)]
        loc = iv - base
        inr = (loc >= 0) & (loc < CHUNK)
        locc = jnp.clip(loc, 0, CHUNK - 1)
        jv = lane + g * LANES
        plsc.store_scatter(tbl_v, [locc], jv, mask=inr)
        t = plsc.load_gather(tbl_v, [locc], mask=inr)
        redo = inr & (jv > t)

        @pl.when(jnp.any(redo))
        def _fixup():
            def fb(k, c2):
                t2 = plsc.load_gather(tbl_v, [locc], mask=inr)
                m2 = inr & (jv > t2)
                plsc.store_scatter(tbl_v, [locc], jv, mask=m2)
                return c2

            lax.fori_loop(0, LANES - 1, fb, 0)

        return carry

    lax.fori_loop(0, GROUPS, body, 0)
    pltpu.sync_copy(tbl_v, t_hbm.at[pl.ds(base, CHUNK)])


@functools.partial(
    pl.kernel,
    out_type=(
        jax.ShapeDtypeStruct((NW, NCH, ICH), jnp.int32),
        jax.ShapeDtypeStruct((B, D), jnp.float32),
        jax.ShapeDtypeStruct((B, D), jnp.float32),
    ),
    mesh=_mesh(),
    compiler_params=_SC_PARAMS,
    scratch_types=[
        pltpu.VMEM((NCH, ICH), jnp.int32),
        pltpu.VMEM((NCH, ICH), jnp.int32),
        pltpu.VMEM((NCH, ICH), jnp.int32),
        pltpu.VMEM((N_PER_W, D), jnp.float32),
        pltpu.VMEM((N_PER_W, D), jnp.float32),
        pltpu.SemaphoreType.DMA,
        pltpu.SemaphoreType.DMA,
        pltpu.SemaphoreType.DMA,
    ],
)
def _gather_rows(t_hbm, mem_hbm, val_hbm, ri_hbm, jr_hbm, a_hbm, b_hbm,
                 ri_v, jr_v, jb_v, a_v, b_v, sem_a, sem_j, sem_b):
    wid = lax.axis_index("s") * NC + lax.axis_index("c")
    base = wid * N_PER_W

    pltpu.sync_copy(ri_hbm.at[wid], ri_v)

    a_copies = [
        pltpu.async_copy(mem_hbm.at[ri_v.at[c]],
                         a_v.at[pl.ds(c * ICH, ICH)], sem_a)
        for c in range(NCH)
    ]
    j_copies = [
        pltpu.async_copy(t_hbm.at[ri_v.at[c]], jr_v.at[c], sem_j)
        for c in range(NCH)
    ]
    for cp in j_copies:
        cp.wait()

    for c in range(NCH):
        for i in range(ICH // LANES):
            v = jr_v[c, pl.ds(i * LANES, LANES)]
            jb_v[c, pl.ds(i * LANES, LANES)] = jnp.maximum(v, 0)

    b_copies = [
        pltpu.async_copy(val_hbm.at[jb_v.at[c]],
                         b_v.at[pl.ds(c * ICH, ICH)], sem_b)
        for c in range(NCH)
    ]
    for cp in b_copies:
        cp.wait()
    for cp in a_copies:
        cp.wait()

    pltpu.sync_copy(jr_v, jr_hbm.at[wid])
    pltpu.sync_copy(a_v, a_hbm.at[pl.ds(base, N_PER_W)])
    pltpu.sync_copy(b_v, b_hbm.at[pl.ds(base, N_PER_W)])


_BLK = 2048


def _math_body(val_ref, a_ref, b_ref, jr_ref, out_ref):
    hit = jr_ref[...] >= 0
    sel = jnp.where(hit, b_ref[...], a_ref[...])
    v = val_ref[...]

    def proj(x):
        n2 = jnp.sum(x * x, axis=-1, keepdims=True)
        n = jnp.sqrt(n2)
        return x * (jnp.tanh(n) / (n + 1e-6))

    q = proj(v)
    read = proj(sel)
    x2 = jnp.sum(q * q, axis=-1, keepdims=True)
    y2 = jnp.sum(read * read, axis=-1, keepdims=True)
    diff2 = jnp.sum((q - read) ** 2, axis=-1, keepdims=True)
    denom = jnp.maximum((1.0 - x2) * (1.0 - y2), 1e-9)
    t = jnp.maximum(2.0 * diff2 / denom, 1e-6)
    dist = jnp.log1p(t + jnp.sqrt(t * (t + 2.0)))
    mem_norm = jnp.sqrt(y2)
    decay = 1.0 - mem_norm
    busemann = dist + mem_norm * (1.0 - decay)
    out_ref[...] = read * jax.nn.sigmoid(-busemann)


def _math(val, a, b, jr):
    grid = (B // _BLK,)
    return pl.pallas_call(
        _math_body,
        grid=grid,
        in_specs=[
            pl.BlockSpec((_BLK, D), lambda i: (i, 0)),
            pl.BlockSpec((_BLK, D), lambda i: (i, 0)),
            pl.BlockSpec((_BLK, D), lambda i: (i, 0)),
            pl.BlockSpec((_BLK, 1), lambda i: (i, 0)),
        ],
        out_specs=pl.BlockSpec((_BLK, D), lambda i: (i, 0)),
        out_shape=jax.ShapeDtypeStruct((B, D), jnp.float32),
    )(val, a, b, jr)


def kernel(mem, idx, val, read_idx):
    idx32 = idx.astype(jnp.int32)
    ri32 = read_idx.astype(jnp.int32)
    t = _build_table(idx32)
    jr3, a, b = _gather_rows(t, mem, val, ri32.reshape(NW, NCH, ICH))
    jr = jr3.reshape(B, 1)
    return _math(val, a, b, jr)
